# Initial kernel scaffold; baseline (speedup 1.0000x reference)
import jax
import jax.numpy as jnp
from jax.experimental import pallas as pl


def kernel(x, edge_index, W_self1, W_neigh1, b1, W_self2, W_neigh2, b2):
    raise NotImplementedError("write your pallas kernel here")


# speedup vs baseline: 4.4814x; 4.4814x over previous
import functools

import jax
import jax.numpy as jnp
from jax import lax
from jax.experimental import pallas as pl
from jax.experimental.pallas import tpu as pltpu
from jax.experimental.pallas import tpu_sc as plsc

NCORES = 2
NSUB = 16
NW = NCORES * NSUB
CHUNK = 128
KCH = 8
DEGW = 16


def _make_sc_segsum(n_pad, d, n_chunks, with_deg):
    mesh = plsc.VectorSubcoreMesh(core_axis_name="c", subcore_axis_name="s")
    rows_per_sub = n_pad // NSUB

    out_type = [jax.ShapeDtypeStruct((NCORES, n_pad, d), jnp.float32)]
    scratch = [
        pltpu.VMEM((KCH, CHUNK), jnp.int32),
        pltpu.VMEM((KCH, CHUNK), jnp.int32),
        pltpu.VMEM((CHUNK, d), jnp.float32),
        pltpu.VMEM_SHARED((n_pad, d), jnp.float32),
    ]
    if with_deg:
        out_type.append(
            jax.ShapeDtypeStruct((NCORES, n_pad, DEGW), jnp.float32))
        scratch += [
            pltpu.VMEM((CHUNK, DEGW), jnp.float32),
            pltpu.VMEM_SHARED((n_pad, DEGW), jnp.float32),
        ]

    def body(table_hbm, src_hbm, dst_hbm, zeros_hbm, zeros16_hbm, ones_hbm,
             agg_out, *rest):
        if with_deg:
            deg_out, src_v, dst_v, rows_v, agg_sh, ones_v, deg_sh = rest
        else:
            src_v, dst_v, rows_v, agg_sh = rest
        c = lax.axis_index("c")
        s = lax.axis_index("s")
        sl = pl.ds(s * rows_per_sub, rows_per_sub)
        pltpu.sync_copy(zeros_hbm.at[sl], agg_sh.at[sl])
        if with_deg:
            pltpu.sync_copy(zeros16_hbm.at[sl], deg_sh.at[sl])
            pltpu.sync_copy(ones_hbm, ones_v)
        plsc.subcore_barrier()

        @pl.loop(0, n_chunks // KCH)
        def _(g):
            pltpu.sync_copy(src_hbm.at[c, s, pl.ds(g * KCH, KCH)], src_v)
            pltpu.sync_copy(dst_hbm.at[c, s, pl.ds(g * KCH, KCH)], dst_v)

            @pl.loop(0, KCH)
            def _(j):
                pltpu.sync_copy(table_hbm.at[src_v.at[j]], rows_v)
                pltpu.sync_copy(rows_v, agg_sh.at[dst_v.at[j]], add=True)
                if with_deg:
                    pltpu.sync_copy(ones_v, deg_sh.at[dst_v.at[j]], add=True)

        plsc.subcore_barrier()
        pltpu.sync_copy(agg_sh.at[sl], agg_out.at[c, sl])
        if with_deg:
            pltpu.sync_copy(deg_sh.at[sl], deg_out.at[c, sl])

    return pl.kernel(body, out_type=tuple(out_type), mesh=mesh,
                     scratch_types=scratch,
                     compiler_params=pltpu.CompilerParams(
                         use_tc_tiling_on_sc=False))


def _mm2_kernel(x_ref, wa_ref, wb_ref, oa_ref, ob_ref):
    x = x_ref[...]
    oa_ref[...] = jnp.dot(x, wa_ref[...], preferred_element_type=jnp.float32)
    ob_ref[...] = jnp.dot(x, wb_ref[...], preferred_element_type=jnp.float32)


def _layer1_kernel(xs_ref, a0_ref, a1_ref, d0_ref, d1_ref, b_ref,
                   ws2_ref, wn2_ref, hs_ref, hn_ref):
    deg = d0_ref[:, 0] + d1_ref[:, 0]
    rdeg = 1.0 / jnp.maximum(deg, 1.0)
    agg = (a0_ref[...] + a1_ref[...]) * rdeg[:, None]
    h = jnp.maximum(xs_ref[...] + agg + b_ref[...], 0.0)
    hs_ref[...] = jnp.dot(h, ws2_ref[...], preferred_element_type=jnp.float32)
    hn_ref[...] = jnp.dot(h, wn2_ref[...], preferred_element_type=jnp.float32)


def _layer2_kernel(hs_ref, a0_ref, a1_ref, d0_ref, d1_ref, b_ref, o_ref):
    deg = d0_ref[:, 0] + d1_ref[:, 0]
    rdeg = 1.0 / jnp.maximum(deg, 1.0)
    agg = (a0_ref[...] + a1_ref[...]) * rdeg[:, None]
    logits = hs_ref[...] + agg + b_ref[...]
    m = jnp.max(logits, axis=-1, keepdims=True)
    z = logits - m
    lse = jnp.log(jnp.sum(jnp.exp(z), axis=-1, keepdims=True))
    o_ref[...] = z - lse


def kernel(x, edge_index, W_self1, W_neigh1, b1, W_self2, W_neigh2, b2):
    n, d_in = x.shape
    d_h = W_self1.shape[1]
    d_out = W_self2.shape[1]
    e = edge_index.shape[1]

    n_pad = ((n + NSUB - 1) // NSUB + 7) // 8 * 8 * NSUB
    e_per_w = e // NW
    group = KCH * CHUNK
    n_chunks = (e_per_w + group - 1) // group * KCH
    e_pad_w = n_chunks * CHUNK

    src = edge_index[0].astype(jnp.int32).reshape(NW, e_per_w)
    dst = edge_index[1].astype(jnp.int32).reshape(NW, e_per_w)
    pad = e_pad_w - e_per_w
    src = jnp.pad(src, ((0, 0), (0, pad)))
    dst = jnp.pad(dst, ((0, 0), (0, pad)), constant_values=n)
    src = src.reshape(NCORES, NSUB, n_chunks, CHUNK)
    dst = dst.reshape(NCORES, NSUB, n_chunks, CHUNK)

    zeros_h = jnp.zeros((n_pad, d_h), jnp.float32)
    zeros_o = jnp.zeros((n_pad, d_out), jnp.float32)
    zeros16 = jnp.zeros((n_pad, DEGW), jnp.float32)
    ones16 = jnp.ones((CHUNK, DEGW), jnp.float32)

    grid_r = 1000
    grid = (n // grid_r,)

    xs1, xn1 = pl.pallas_call(
        _mm2_kernel,
        grid=grid,
        in_specs=[
            pl.BlockSpec((grid_r, d_in), lambda i: (i, 0)),
            pl.BlockSpec((d_in, d_h), lambda i: (0, 0)),
            pl.BlockSpec((d_in, d_h), lambda i: (0, 0)),
        ],
        out_specs=[
            pl.BlockSpec((grid_r, d_h), lambda i: (i, 0)),
            pl.BlockSpec((grid_r, d_h), lambda i: (i, 0)),
        ],
        out_shape=[
            jax.ShapeDtypeStruct((n, d_h), jnp.float32),
            jax.ShapeDtypeStruct((n, d_h), jnp.float32),
        ],
    )(x, W_self1, W_neigh1)

    agg1p, degp = _make_sc_segsum(n_pad, d_h, n_chunks, True)(
        xn1, src, dst, zeros_h, zeros16, ones16)

    hs2, hn2 = pl.pallas_call(
        _layer1_kernel,
        grid=grid,
        in_specs=[
            pl.BlockSpec((grid_r, d_h), lambda i: (i, 0)),
            pl.BlockSpec((None, grid_r, d_h), lambda i: (0, i, 0)),
            pl.BlockSpec((None, grid_r, d_h), lambda i: (1, i, 0)),
            pl.BlockSpec((None, grid_r, DEGW), lambda i: (0, i, 0)),
            pl.BlockSpec((None, grid_r, DEGW), lambda i: (1, i, 0)),
            pl.BlockSpec((1, d_h), lambda i: (0, 0)),
            pl.BlockSpec((d_h, d_out), lambda i: (0, 0)),
            pl.BlockSpec((d_h, d_out), lambda i: (0, 0)),
        ],
        out_specs=[
            pl.BlockSpec((grid_r, d_out), lambda i: (i, 0)),
            pl.BlockSpec((grid_r, d_out), lambda i: (i, 0)),
        ],
        out_shape=[
            jax.ShapeDtypeStruct((n, d_out), jnp.float32),
            jax.ShapeDtypeStruct((n, d_out), jnp.float32),
        ],
    )(xs1, agg1p, agg1p, degp, degp, b1.reshape(1, d_h), W_self2, W_neigh2)

    (agg2p,) = _make_sc_segsum(n_pad, d_out, n_chunks, False)(
        hn2, src, dst, zeros_o, zeros16, ones16)

    out = pl.pallas_call(
        _layer2_kernel,
        grid=grid,
        in_specs=[
            pl.BlockSpec((grid_r, d_out), lambda i: (i, 0)),
            pl.BlockSpec((None, grid_r, d_out), lambda i: (0, i, 0)),
            pl.BlockSpec((None, grid_r, d_out), lambda i: (1, i, 0)),
            pl.BlockSpec((None, grid_r, DEGW), lambda i: (0, i, 0)),
            pl.BlockSpec((None, grid_r, DEGW), lambda i: (1, i, 0)),
            pl.BlockSpec((1, d_out), lambda i: (0, 0)),
        ],
        out_specs=pl.BlockSpec((grid_r, d_out), lambda i: (i, 0)),
        out_shape=jax.ShapeDtypeStruct((n, d_out), jnp.float32),
    )(hs2, agg2p, agg2p, degp, degp, b2.reshape(1, d_out))

    return out


# speedup vs baseline: 4.9089x; 1.0954x over previous
import functools

import jax
import jax.numpy as jnp
from jax import lax
from jax.experimental import pallas as pl
from jax.experimental.pallas import tpu as pltpu
from jax.experimental.pallas import tpu_sc as plsc

NCORES = 2
NSUB = 16
NW = NCORES * NSUB
CHUNK = 128
KCH = 8
DEGW = 16


_SC_MESH = plsc.VectorSubcoreMesh(core_axis_name="c", subcore_axis_name="s")
_SC_PARAMS = pltpu.CompilerParams(use_tc_tiling_on_sc=False)


def _make_sc_segsum(n_pad, d, n_chunks):
    rows_per_sub = n_pad // NSUB

    out_type = jax.ShapeDtypeStruct((NCORES, n_pad, d), jnp.float32)
    scratch = [
        pltpu.VMEM((KCH, CHUNK), jnp.int32),
        pltpu.VMEM((KCH, CHUNK), jnp.int32),
        pltpu.VMEM((CHUNK, d), jnp.float32),
        pltpu.VMEM((CHUNK, d), jnp.float32),
        pltpu.VMEM_SHARED((n_pad, d), jnp.float32),
        pltpu.SemaphoreType.DMA,
        pltpu.SemaphoreType.DMA,
    ]

    def body(table_hbm, src_hbm, dst_hbm, zeros_hbm, agg_out,
             src_v, dst_v, rows0, rows1, agg_sh, gsem, ssem):
        rows = (rows0, rows1)
        c = lax.axis_index("c")
        s = lax.axis_index("s")
        sl = pl.ds(s * rows_per_sub, rows_per_sub)
        pltpu.sync_copy(zeros_hbm.at[sl], agg_sh.at[sl])
        plsc.subcore_barrier()

        @pl.loop(0, n_chunks // KCH)
        def _(g):
            pltpu.sync_copy(src_hbm.at[c, s, pl.ds(g * KCH, KCH)], src_v)
            pltpu.sync_copy(dst_hbm.at[c, s, pl.ds(g * KCH, KCH)], dst_v)
            gathers = [
                pltpu.async_copy(table_hbm.at[src_v.at[j]], rows[j % 2], gsem)
                for j in (0,)]
            scatters = []
            for j in range(KCH):
                gathers[j].wait()
                scatters.append(pltpu.async_copy(
                    rows[j % 2], agg_sh.at[dst_v.at[j]], ssem, add=True))
                if j + 1 < KCH:
                    if j >= 1:
                        scatters[j - 1].wait()
                    gathers.append(pltpu.async_copy(
                        table_hbm.at[src_v.at[j + 1]], rows[(j + 1) % 2],
                        gsem))
            scatters[KCH - 2].wait()
            scatters[KCH - 1].wait()

        plsc.subcore_barrier()
        pltpu.sync_copy(agg_sh.at[sl], agg_out.at[c, sl])

    return pl.kernel(body, out_type=out_type, mesh=_SC_MESH,
                     scratch_types=scratch, compiler_params=_SC_PARAMS)


def _make_sc_deg(n_pad, n_chunks):
    rows_per_sub = n_pad // NSUB

    out_type = jax.ShapeDtypeStruct((NCORES, n_pad, DEGW), jnp.float32)
    scratch = [
        pltpu.VMEM((KCH, CHUNK), jnp.int32),
        pltpu.VMEM((CHUNK, DEGW), jnp.float32),
        pltpu.VMEM_SHARED((n_pad, DEGW), jnp.float32),
        pltpu.SemaphoreType.DMA,
    ]

    def body(dst_hbm, zeros16_hbm, ones_hbm, deg_out,
             dst_v, ones_v, deg_sh, ssem):
        c = lax.axis_index("c")
        s = lax.axis_index("s")
        sl = pl.ds(s * rows_per_sub, rows_per_sub)
        pltpu.sync_copy(zeros16_hbm.at[sl], deg_sh.at[sl])
        pltpu.sync_copy(ones_hbm, ones_v)
        plsc.subcore_barrier()

        @pl.loop(0, n_chunks // KCH)
        def _(g):
            pltpu.sync_copy(dst_hbm.at[c, s, pl.ds(g * KCH, KCH)], dst_v)
            scatters = [
                pltpu.async_copy(ones_v, deg_sh.at[dst_v.at[j]], ssem,
                                 add=True)
                for j in range(KCH)]
            for sc in scatters:
                sc.wait()

        plsc.subcore_barrier()
        pltpu.sync_copy(deg_sh.at[sl], deg_out.at[c, sl])

    return pl.kernel(body, out_type=out_type, mesh=_SC_MESH,
                     scratch_types=scratch, compiler_params=_SC_PARAMS)


def _mm2_kernel(x_ref, wa_ref, wb_ref, oa_ref, ob_ref):
    x = x_ref[...]
    oa_ref[...] = jnp.dot(x, wa_ref[...], preferred_element_type=jnp.float32)
    ob_ref[...] = jnp.dot(x, wb_ref[...], preferred_element_type=jnp.float32)


def _layer1_kernel(xs_ref, a0_ref, a1_ref, d0_ref, d1_ref, b_ref,
                   ws2_ref, wn2_ref, hs_ref, hn_ref):
    deg = d0_ref[:, 0] + d1_ref[:, 0]
    rdeg = 1.0 / jnp.maximum(deg, 1.0)
    agg = (a0_ref[...] + a1_ref[...]) * rdeg[:, None]
    h = jnp.maximum(xs_ref[...] + agg + b_ref[...], 0.0)
    hs_ref[...] = jnp.dot(h, ws2_ref[...], preferred_element_type=jnp.float32)
    hn_ref[...] = jnp.dot(h, wn2_ref[...], preferred_element_type=jnp.float32)


def _layer2_kernel(hs_ref, a0_ref, a1_ref, d0_ref, d1_ref, b_ref, o_ref):
    deg = d0_ref[:, 0] + d1_ref[:, 0]
    rdeg = 1.0 / jnp.maximum(deg, 1.0)
    agg = (a0_ref[...] + a1_ref[...]) * rdeg[:, None]
    logits = hs_ref[...] + agg + b_ref[...]
    m = jnp.max(logits, axis=-1, keepdims=True)
    z = logits - m
    lse = jnp.log(jnp.sum(jnp.exp(z), axis=-1, keepdims=True))
    o_ref[...] = z - lse


def kernel(x, edge_index, W_self1, W_neigh1, b1, W_self2, W_neigh2, b2):
    n, d_in = x.shape
    d_h = W_self1.shape[1]
    d_out = W_self2.shape[1]
    e = edge_index.shape[1]

    n_pad = ((n + NSUB - 1) // NSUB + 7) // 8 * 8 * NSUB
    e_per_w = e // NW
    group = KCH * CHUNK
    n_chunks = (e_per_w + group - 1) // group * KCH
    e_pad_w = n_chunks * CHUNK

    src = edge_index[0].astype(jnp.int32).reshape(NW, e_per_w)
    dst = edge_index[1].astype(jnp.int32).reshape(NW, e_per_w)
    pad = e_pad_w - e_per_w
    src = jnp.pad(src, ((0, 0), (0, pad)))
    dst = jnp.pad(dst, ((0, 0), (0, pad)), constant_values=n)
    src = src.reshape(NCORES, NSUB, n_chunks, CHUNK)
    dst = dst.reshape(NCORES, NSUB, n_chunks, CHUNK)

    zeros_h = jnp.zeros((n_pad, d_h), jnp.float32)
    zeros_o = jnp.zeros((n_pad, d_out), jnp.float32)
    zeros16 = jnp.zeros((n_pad, DEGW), jnp.float32)
    ones16 = jnp.ones((CHUNK, DEGW), jnp.float32)

    grid_r = 1000
    grid = (n // grid_r,)

    xs1, xn1 = pl.pallas_call(
        _mm2_kernel,
        grid=grid,
        in_specs=[
            pl.BlockSpec((grid_r, d_in), lambda i: (i, 0)),
            pl.BlockSpec((d_in, d_h), lambda i: (0, 0)),
            pl.BlockSpec((d_in, d_h), lambda i: (0, 0)),
        ],
        out_specs=[
            pl.BlockSpec((grid_r, d_h), lambda i: (i, 0)),
            pl.BlockSpec((grid_r, d_h), lambda i: (i, 0)),
        ],
        out_shape=[
            jax.ShapeDtypeStruct((n, d_h), jnp.float32),
            jax.ShapeDtypeStruct((n, d_h), jnp.float32),
        ],
    )(x, W_self1, W_neigh1)

    degp = _make_sc_deg(n_pad, n_chunks)(dst, zeros16, ones16)

    agg1p = _make_sc_segsum(n_pad, d_h, n_chunks)(xn1, src, dst, zeros_h)

    hs2, hn2 = pl.pallas_call(
        _layer1_kernel,
        grid=grid,
        in_specs=[
            pl.BlockSpec((grid_r, d_h), lambda i: (i, 0)),
            pl.BlockSpec((None, grid_r, d_h), lambda i: (0, i, 0)),
            pl.BlockSpec((None, grid_r, d_h), lambda i: (1, i, 0)),
            pl.BlockSpec((None, grid_r, DEGW), lambda i: (0, i, 0)),
            pl.BlockSpec((None, grid_r, DEGW), lambda i: (1, i, 0)),
            pl.BlockSpec((1, d_h), lambda i: (0, 0)),
            pl.BlockSpec((d_h, d_out), lambda i: (0, 0)),
            pl.BlockSpec((d_h, d_out), lambda i: (0, 0)),
        ],
        out_specs=[
            pl.BlockSpec((grid_r, d_out), lambda i: (i, 0)),
            pl.BlockSpec((grid_r, d_out), lambda i: (i, 0)),
        ],
        out_shape=[
            jax.ShapeDtypeStruct((n, d_out), jnp.float32),
            jax.ShapeDtypeStruct((n, d_out), jnp.float32),
        ],
    )(xs1, agg1p, agg1p, degp, degp, b1.reshape(1, d_h), W_self2, W_neigh2)

    agg2p = _make_sc_segsum(n_pad, d_out, n_chunks)(hn2, src, dst, zeros_o)

    out = pl.pallas_call(
        _layer2_kernel,
        grid=grid,
        in_specs=[
            pl.BlockSpec((grid_r, d_out), lambda i: (i, 0)),
            pl.BlockSpec((None, grid_r, d_out), lambda i: (0, i, 0)),
            pl.BlockSpec((None, grid_r, d_out), lambda i: (1, i, 0)),
            pl.BlockSpec((None, grid_r, DEGW), lambda i: (0, i, 0)),
            pl.BlockSpec((None, grid_r, DEGW), lambda i: (1, i, 0)),
            pl.BlockSpec((1, d_out), lambda i: (0, 0)),
        ],
        out_specs=pl.BlockSpec((grid_r, d_out), lambda i: (i, 0)),
        out_shape=jax.ShapeDtypeStruct((n, d_out), jnp.float32),
    )(hs2, agg2p, agg2p, degp, degp, b2.reshape(1, d_out))

    return out


# speedup vs baseline: 9.7411x; 1.9844x over previous
import functools

import jax
import jax.numpy as jnp
from jax import lax
from jax.experimental import pallas as pl
from jax.experimental.pallas import tpu as pltpu
from jax.experimental.pallas import tpu_sc as plsc

NCORES = 2
NSUB = 16
NW = NCORES * NSUB
CHUNK = 128
KCH = 8
DEGW = 16


_SC_MESH = plsc.VectorSubcoreMesh(core_axis_name="c", subcore_axis_name="s")
_SC_PARAMS = pltpu.CompilerParams(use_tc_tiling_on_sc=False)


def _make_sc_segsum(n, n_pad, d, n_chunks, col_split):
    rows_per_sub = n_pad // NSUB
    trows_per_sub = n // NSUB

    out_type = jax.ShapeDtypeStruct((NCORES, n_pad, d), jnp.float32)
    scratch = [
        pltpu.VMEM((KCH, CHUNK), jnp.int32),
        pltpu.VMEM((KCH, CHUNK), jnp.int32),
        pltpu.VMEM((CHUNK, d), jnp.float32),
        pltpu.VMEM((CHUNK, d), jnp.float32),
        pltpu.VMEM_SHARED((n, d), jnp.float32),
        pltpu.VMEM_SHARED((n_pad, d), jnp.float32),
        pltpu.SemaphoreType.DMA,
        pltpu.SemaphoreType.DMA,
    ]

    def body(table_hbm, src_hbm, dst_hbm, zeros_hbm, agg_out,
             src_v, dst_v, rows0, rows1, table_sh, agg_sh, gsem, ssem):
        rows = (rows0, rows1)
        c = lax.axis_index("c")
        s = lax.axis_index("s")
        sl = pl.ds(s * rows_per_sub, rows_per_sub)
        tsl = pl.ds(s * trows_per_sub, trows_per_sub)
        if col_split:
            pltpu.sync_copy(table_hbm.at[tsl, pl.ds(c * d, d)],
                            table_sh.at[tsl])
        else:
            pltpu.sync_copy(table_hbm.at[tsl], table_sh.at[tsl])
        pltpu.sync_copy(zeros_hbm.at[sl], agg_sh.at[sl])
        plsc.subcore_barrier()

        @pl.loop(0, n_chunks // KCH)
        def _(g):
            if col_split:
                pltpu.sync_copy(src_hbm.at[s, pl.ds(g * KCH, KCH)], src_v)
                pltpu.sync_copy(dst_hbm.at[s, pl.ds(g * KCH, KCH)], dst_v)
            else:
                pltpu.sync_copy(src_hbm.at[c, s, pl.ds(g * KCH, KCH)], src_v)
                pltpu.sync_copy(dst_hbm.at[c, s, pl.ds(g * KCH, KCH)], dst_v)
            gathers = [
                pltpu.async_copy(table_sh.at[src_v.at[j]], rows[j % 2], gsem)
                for j in (0,)]
            scatters = []
            for j in range(KCH):
                gathers[j].wait()
                scatters.append(pltpu.async_copy(
                    rows[j % 2], agg_sh.at[dst_v.at[j]], ssem, add=True))
                if j + 1 < KCH:
                    if j >= 1:
                        scatters[j - 1].wait()
                    gathers.append(pltpu.async_copy(
                        table_sh.at[src_v.at[j + 1]], rows[(j + 1) % 2],
                        gsem))
            scatters[KCH - 2].wait()
            scatters[KCH - 1].wait()

        plsc.subcore_barrier()
        pltpu.sync_copy(agg_sh.at[sl], agg_out.at[c, sl])

    return pl.kernel(body, out_type=out_type, mesh=_SC_MESH,
                     scratch_types=scratch, compiler_params=_SC_PARAMS)


def _make_sc_deg(n_pad, n_chunks):
    rows_per_sub = n_pad // NSUB

    out_type = jax.ShapeDtypeStruct((NCORES, n_pad, DEGW), jnp.float32)
    scratch = [
        pltpu.VMEM((KCH, CHUNK), jnp.int32),
        pltpu.VMEM((CHUNK, DEGW), jnp.float32),
        pltpu.VMEM_SHARED((n_pad, DEGW), jnp.float32),
        pltpu.SemaphoreType.DMA,
    ]

    def body(dst_hbm, zeros16_hbm, ones_hbm, deg_out,
             dst_v, ones_v, deg_sh, ssem):
        c = lax.axis_index("c")
        s = lax.axis_index("s")
        sl = pl.ds(s * rows_per_sub, rows_per_sub)
        pltpu.sync_copy(zeros16_hbm.at[sl], deg_sh.at[sl])
        pltpu.sync_copy(ones_hbm, ones_v)
        plsc.subcore_barrier()

        @pl.loop(0, n_chunks // KCH)
        def _(g):
            pltpu.sync_copy(dst_hbm.at[c, s, pl.ds(g * KCH, KCH)], dst_v)
            scatters = [
                pltpu.async_copy(ones_v, deg_sh.at[dst_v.at[j]], ssem,
                                 add=True)
                for j in range(KCH)]
            for sc in scatters:
                sc.wait()

        plsc.subcore_barrier()
        pltpu.sync_copy(deg_sh.at[sl], deg_out.at[c, sl])

    return pl.kernel(body, out_type=out_type, mesh=_SC_MESH,
                     scratch_types=scratch, compiler_params=_SC_PARAMS)


def _mm2_kernel(x_ref, wa_ref, wb_ref, oa_ref, ob_ref):
    x = x_ref[...]
    oa_ref[...] = jnp.dot(x, wa_ref[...], preferred_element_type=jnp.float32)
    ob_ref[...] = jnp.dot(x, wb_ref[...], preferred_element_type=jnp.float32)


def _layer1_kernel(xs_ref, a0_ref, a1_ref, d0_ref, d1_ref, b_ref,
                   ws2_ref, wn2_ref, hs_ref, hn_ref):
    deg = d0_ref[:, 0] + d1_ref[:, 0]
    rdeg = 1.0 / jnp.maximum(deg, 1.0)
    agg = jnp.concatenate([a0_ref[...], a1_ref[...]], axis=1) * rdeg[:, None]
    h = jnp.maximum(xs_ref[...] + agg + b_ref[...], 0.0)
    hs_ref[...] = jnp.dot(h, ws2_ref[...], preferred_element_type=jnp.float32)
    hn_ref[...] = jnp.dot(h, wn2_ref[...], preferred_element_type=jnp.float32)


def _layer2_kernel(hs_ref, a0_ref, a1_ref, d0_ref, d1_ref, b_ref, o_ref):
    deg = d0_ref[:, 0] + d1_ref[:, 0]
    rdeg = 1.0 / jnp.maximum(deg, 1.0)
    agg = (a0_ref[...] + a1_ref[...]) * rdeg[:, None]
    logits = hs_ref[...] + agg + b_ref[...]
    m = jnp.max(logits, axis=-1, keepdims=True)
    z = logits - m
    lse = jnp.log(jnp.sum(jnp.exp(z), axis=-1, keepdims=True))
    o_ref[...] = z - lse


def kernel(x, edge_index, W_self1, W_neigh1, b1, W_self2, W_neigh2, b2):
    n, d_in = x.shape
    d_h = W_self1.shape[1]
    d_out = W_self2.shape[1]
    e = edge_index.shape[1]

    n_pad = ((n + NSUB - 1) // NSUB + 7) // 8 * 8 * NSUB
    group = KCH * CHUNK
    src_flat = edge_index[0].astype(jnp.int32)
    dst_flat = edge_index[1].astype(jnp.int32)

    def _partition(nw):
        e_per = e // nw
        n_chunks = (e_per + group - 1) // group * KCH
        pad = n_chunks * CHUNK - e_per
        s = jnp.pad(src_flat.reshape(nw, e_per), ((0, 0), (0, pad)))
        d = jnp.pad(dst_flat.reshape(nw, e_per), ((0, 0), (0, pad)),
                    constant_values=n)
        return (s.reshape(nw, n_chunks, CHUNK),
                d.reshape(nw, n_chunks, CHUNK), n_chunks)

    src1, dst1, nc1 = _partition(NSUB)
    src2, dst2, nc2 = _partition(NW)
    src2 = src2.reshape(NCORES, NSUB, nc2, CHUNK)
    dst2 = dst2.reshape(NCORES, NSUB, nc2, CHUNK)

    d_half = d_h // NCORES
    zeros_hh = jnp.zeros((n_pad, d_half), jnp.float32)
    zeros_o = jnp.zeros((n_pad, d_out), jnp.float32)
    zeros16 = jnp.zeros((n_pad, DEGW), jnp.float32)
    ones16 = jnp.ones((CHUNK, DEGW), jnp.float32)

    grid_r = 1000
    grid = (n // grid_r,)

    xs1, xn1 = pl.pallas_call(
        _mm2_kernel,
        grid=grid,
        in_specs=[
            pl.BlockSpec((grid_r, d_in), lambda i: (i, 0)),
            pl.BlockSpec((d_in, d_h), lambda i: (0, 0)),
            pl.BlockSpec((d_in, d_h), lambda i: (0, 0)),
        ],
        out_specs=[
            pl.BlockSpec((grid_r, d_h), lambda i: (i, 0)),
            pl.BlockSpec((grid_r, d_h), lambda i: (i, 0)),
        ],
        out_shape=[
            jax.ShapeDtypeStruct((n, d_h), jnp.float32),
            jax.ShapeDtypeStruct((n, d_h), jnp.float32),
        ],
    )(x, W_self1, W_neigh1)

    degp = _make_sc_deg(n_pad, nc2)(dst2, zeros16, ones16)

    agg1p = _make_sc_segsum(n, n_pad, d_half, nc1, True)(
        xn1, src1, dst1, zeros_hh)

    hs2, hn2 = pl.pallas_call(
        _layer1_kernel,
        grid=grid,
        in_specs=[
            pl.BlockSpec((grid_r, d_h), lambda i: (i, 0)),
            pl.BlockSpec((None, grid_r, d_half), lambda i: (0, i, 0)),
            pl.BlockSpec((None, grid_r, d_half), lambda i: (1, i, 0)),
            pl.BlockSpec((None, grid_r, DEGW), lambda i: (0, i, 0)),
            pl.BlockSpec((None, grid_r, DEGW), lambda i: (1, i, 0)),
            pl.BlockSpec((1, d_h), lambda i: (0, 0)),
            pl.BlockSpec((d_h, d_out), lambda i: (0, 0)),
            pl.BlockSpec((d_h, d_out), lambda i: (0, 0)),
        ],
        out_specs=[
            pl.BlockSpec((grid_r, d_out), lambda i: (i, 0)),
            pl.BlockSpec((grid_r, d_out), lambda i: (i, 0)),
        ],
        out_shape=[
            jax.ShapeDtypeStruct((n, d_out), jnp.float32),
            jax.ShapeDtypeStruct((n, d_out), jnp.float32),
        ],
    )(xs1, agg1p, agg1p, degp, degp, b1.reshape(1, d_h), W_self2, W_neigh2)

    agg2p = _make_sc_segsum(n, n_pad, d_out, nc2, False)(
        hn2, src2, dst2, zeros_o)

    out = pl.pallas_call(
        _layer2_kernel,
        grid=grid,
        in_specs=[
            pl.BlockSpec((grid_r, d_out), lambda i: (i, 0)),
            pl.BlockSpec((None, grid_r, d_out), lambda i: (0, i, 0)),
            pl.BlockSpec((None, grid_r, d_out), lambda i: (1, i, 0)),
            pl.BlockSpec((None, grid_r, DEGW), lambda i: (0, i, 0)),
            pl.BlockSpec((None, grid_r, DEGW), lambda i: (1, i, 0)),
            pl.BlockSpec((1, d_out), lambda i: (0, 0)),
        ],
        out_specs=pl.BlockSpec((grid_r, d_out), lambda i: (i, 0)),
        out_shape=jax.ShapeDtypeStruct((n, d_out), jnp.float32),
    )(hs2, agg2p, agg2p, degp, degp, b2.reshape(1, d_out))

    return out
